# baseline (speedup 1.0000x reference)
import functools

import jax
import jax.numpy as jnp
from jax import lax
from jax.experimental import pallas as pl
from jax.experimental.pallas import tpu as pltpu
from jax.experimental.pallas import tpu_sc as plsc

D = 128
TOPK = 10
K3 = 3
NCORES = 2
NSUB = 16
NW = NCORES * NSUB
CH = 128
NBUF = 4
NPAD = 4096
ZR = 10000
NT = 10016


def _faug_body(xu_ref, ex_ref, wa_ref, wb_ref, o_ref):
    o_ref[...] = (
        jnp.dot(xu_ref[...], wa_ref[...], preferred_element_type=jnp.float32)
        + jnp.dot(ex_ref[...], wb_ref[...], preferred_element_type=jnp.float32)
    )


def _tc_faug(xu, ex, wa, wb):
    n = xu.shape[0]
    return pl.pallas_call(
        _faug_body,
        out_shape=jax.ShapeDtypeStruct((n, D), jnp.float32),
    )(xu, ex, wa, wb)


def _topk_body(fb_ref, fa_ref, o_ref, *, k, br, n):
    i = pl.program_id(0)
    s = lax.dot_general(fb_ref[...], fa_ref[...], (((1,), (1,)), ((), ())),
                        preferred_element_type=jnp.float32)
    rows = lax.broadcasted_iota(jnp.int32, (br, n), 0) + i * br
    cols = lax.broadcasted_iota(jnp.int32, (br, n), 1)
    s = jnp.where(cols == rows, s - 1e9, s)
    outs = []
    for _ in range(k):
        m = jnp.max(s, axis=1, keepdims=True)
        idx = jnp.min(jnp.where(s == m, cols, n), axis=1)
        outs.append(idx[:, None])
        s = jnp.where(cols == idx[:, None], -2e9, s)
    o_ref[...] = jnp.concatenate(outs, axis=1)


def _tc_topk(f, k, br=400):
    n = f.shape[0]
    grid = (n // br,)
    return pl.pallas_call(
        functools.partial(_topk_body, k=k, br=br, n=n),
        grid=grid,
        in_specs=[
            pl.BlockSpec((br, D), lambda i: (i, 0)),
            pl.BlockSpec((n, D), lambda i: (0, 0)),
        ],
        out_specs=pl.BlockSpec((br, k), lambda i: (i, 0)),
        out_shape=jax.ShapeDtypeStruct((n, k), jnp.int32),
    )(f, f)


def _topkg_body(fb_ref, fa_ref, a_ref, b_ref, o_ref, d1_ref, d2_ref,
                *, k, br, n, dump):
    i = pl.program_id(0)
    s = lax.dot_general(fb_ref[...], fa_ref[...], (((1,), (1,)), ((), ())),
                        preferred_element_type=jnp.float32)
    rows = lax.broadcasted_iota(jnp.int32, (br, n), 0) + i * br
    cols = lax.broadcasted_iota(jnp.int32, (br, n), 1)
    s = jnp.where(cols == rows, s - 1e9, s)
    a_blk = a_ref[0, 0, :]
    b_all = b_ref[...]
    rvec = lax.broadcasted_iota(jnp.int32, (br,), 0) + i * br
    outs, d1s, d2s = [], [], []
    for _ in range(k):
        m = jnp.max(s, axis=1, keepdims=True)
        idx = jnp.min(jnp.where(s == m, cols, n), axis=1)
        sel = cols == idx[:, None]
        bval = jnp.max(jnp.where(sel, b_all[None, :], -3e38), axis=1)
        g = (a_blk + bval) > 0.0
        outs.append(idx[:, None])
        d1s.append(jnp.where(g, rvec, dump)[:, None])
        d2s.append(jnp.where(g, idx, dump)[:, None])
        s = jnp.where(sel, -2e9, s)
    o_ref[...] = jnp.concatenate(outs, axis=1)
    d1_ref[...] = jnp.concatenate(d1s, axis=1)
    d2_ref[...] = jnp.concatenate(d2s, axis=1)


def _tc_topk_gate(f, a, b, k, dump, br=400):
    n = f.shape[0]
    grid = (n // br,)
    ospec = pl.BlockSpec((br, k), lambda i: (i, 0))
    oshape = jax.ShapeDtypeStruct((n, k), jnp.int32)
    return pl.pallas_call(
        functools.partial(_topkg_body, k=k, br=br, n=n, dump=dump),
        grid=grid,
        in_specs=[
            pl.BlockSpec((br, D), lambda i: (i, 0)),
            pl.BlockSpec((n, D), lambda i: (0, 0)),
            pl.BlockSpec((1, 1, br), lambda i: (i, 0, 0)),
            pl.BlockSpec((n,), lambda i: (0,)),
        ],
        out_specs=(ospec, ospec, ospec),
        out_shape=(oshape, oshape, oshape),
    )(f, f, a.reshape(n // br, 1, br), b)


def _scaleu_body(xu_ref, cu_ref, cs_ref, xsu_ref, xss_ref, rsu_ref, rss_ref,
                 *, nu, npu):
    xu = xu_ref[...]
    degu = cu_ref[0, :npu, 0] + cu_ref[1, :npu, 0] + 1.0
    rsu = lax.rsqrt(degu)
    rsu_ref[...] = rsu
    xsu_ref[...] = xu * rsu[:nu, None]
    degs = cs_ref[0, :npu, 0] + cs_ref[1, :npu, 0] + 1.0
    rss = lax.rsqrt(degs)
    rss_ref[...] = rss
    xss_ref[...] = xu * rss[:nu, None]


def _tc_scaleu(xu, cnt_u, cnt_s, npad_u):
    nu = xu.shape[0]
    return pl.pallas_call(
        functools.partial(_scaleu_body, nu=nu, npu=npad_u),
        out_shape=(
            jax.ShapeDtypeStruct((nu, D), jnp.float32),
            jax.ShapeDtypeStruct((nu, D), jnp.float32),
            jax.ShapeDtypeStruct((npad_u,), jnp.float32),
            jax.ShapeDtypeStruct((npad_u,), jnp.float32),
        ),
    )(xu, cnt_u, cnt_s)


def _fuse_body(pu_ref, ps_ref, xsu_ref, xss_ref, rsu_ref, rss_ref,
               wc1_ref, bc1_ref, wc2_ref, bc2_ref, wt_ref, bt_ref, wg_ref,
               xf_ref, ab_ref, *, nu):
    rsu = rsu_ref[...][:nu]
    aggu = rsu[:, None] * (pu_ref[0, :nu, :] + pu_ref[1, :nu, :] + xsu_ref[...])
    h1 = jax.nn.relu(
        jnp.dot(aggu, wc1_ref[...], preferred_element_type=jnp.float32)
        + bc1_ref[...][None, :])
    rss = rss_ref[...][:nu]
    aggs = rss[:, None] * (ps_ref[0, :nu, :] + ps_ref[1, :nu, :] + xss_ref[...])
    h2 = jax.nn.relu(
        jnp.dot(aggs, wc2_ref[...], preferred_element_type=jnp.float32)
        + bc2_ref[...][None, :])
    hd = wc1_ref.shape[1]
    xf = (jnp.dot(h1, wt_ref[...][:hd], preferred_element_type=jnp.float32)
          + jnp.dot(h2, wt_ref[...][hd:], preferred_element_type=jnp.float32)
          + bt_ref[...][None, :])
    xf_ref[...] = xf
    ab_ref[...] = jnp.dot(xf, wg_ref[...], preferred_element_type=jnp.float32)


def _tc_fuse(p_u, p_s, xs_u, xs_s, rs_u, rs_s, wc1, bc1, wc2, bc2, wt, bt, wg8):
    nu = xs_u.shape[0]
    return pl.pallas_call(
        functools.partial(_fuse_body, nu=nu),
        out_shape=(
            jax.ShapeDtypeStruct((nu, D), jnp.float32),
            jax.ShapeDtypeStruct((nu, 8), jnp.float32),
        ),
    )(p_u, p_s, xs_u, xs_s, rs_u, rs_s, wc1, bc1, wc2, bc2, wt, bt, wg8)


def _prep_body(x_ref, xf_ref, cu_ref, ca_ref, cb_ref, xs0_ref, rs_ref,
               *, nu, nn, nb):
    na = NPAD
    deg = jnp.concatenate(
        [cu_ref[0, :nu, 0] + cu_ref[1, :nu, 0],
         ca_ref[0, :na, 0] + ca_ref[1, :na, 0],
         cb_ref[0, :nb, 0] + cb_ref[1, :nb, 0]]) + 1.0
    rs = lax.rsqrt(deg)
    rs_ref[...] = rs
    xs0_ref[:nu, :] = xf_ref[...] * rs[:nu, None]
    xs0_ref[nu:nn, :] = x_ref[nu:, :] * rs[nu:, None]
    xs0_ref[nn:, :] = jnp.zeros((NT - nn, D), jnp.float32)


def _tc_prep(x, xf, cnt_usr, cnt_itmA, cnt_itmB):
    nn = x.shape[0]
    nu = xf.shape[0]
    nb = nn - nu - NPAD
    return pl.pallas_call(
        functools.partial(_prep_body, nu=nu, nn=nn, nb=nb),
        out_shape=(
            jax.ShapeDtypeStruct((NT, D), jnp.float32),
            jax.ShapeDtypeStruct((nn,), jnp.float32),
        ),
    )(x, xf, cnt_usr, cnt_itmA, cnt_itmB)


def _layer_body(pu_ref, pa_ref, pb_ref, xs_ref, rs_ref, w_ref, b_ref, o_ref,
                *, nn, final, nu):
    nb = nn - nu - NPAD
    p = jnp.concatenate(
        [pu_ref[0, :nu, :] + pu_ref[1, :nu, :],
         pa_ref[0, :, :] + pa_ref[1, :, :],
         pb_ref[0, :nb, :] + pb_ref[1, :nb, :]], axis=0)
    agg = rs_ref[...][:, None] * (p + xs_ref[:nn, :])
    h = (jnp.dot(agg, w_ref[...], preferred_element_type=jnp.float32)
         + b_ref[...][None, :])
    if final:
        o_ref[...] = jax.nn.sigmoid(h[:nu])
    else:
        o_ref[:nn, :] = rs_ref[...][:, None] * jax.nn.relu(h)
        o_ref[nn:, :] = jnp.zeros((NT - nn, h.shape[1]), jnp.float32)


def _tc_layer(p_usr, p_itmA, p_itmB, xs, rs, w, b, final=False, nu=0):
    nn = rs.shape[0]
    dout = w.shape[1]
    oshape = (nu, dout) if final else (NT, dout)
    return pl.pallas_call(
        functools.partial(_layer_body, nn=nn, final=final, nu=nu),
        out_shape=jax.ShapeDtypeStruct(oshape, jnp.float32),
    )(p_usr, p_itmA, p_itmB, xs, rs, w, b)


def _mesh():
    return plsc.VectorSubcoreMesh(core_axis_name="c", subcore_axis_name="s")


def _make_agg(ncht, npad, d):
    rpt = npad // NSUB

    @functools.partial(
        pl.kernel,
        out_type=jax.ShapeDtypeStruct((NCORES, npad, d), jnp.float32),
        mesh=_mesh(),
        scratch_types=[
            pltpu.VMEM((ncht, CH), jnp.int32),
            pltpu.VMEM((ncht, CH), jnp.int32),
            pltpu.VMEM((NBUF, CH, d), jnp.float32),
            pltpu.VMEM((16,), jnp.int32),
            pltpu.VMEM_SHARED((npad, d), jnp.float32),
            [pltpu.SemaphoreType.DMA] * NBUF,
            [pltpu.SemaphoreType.DMA] * NBUF,
        ],
    )
    def k(src_hbm, dst_hbm, tab_hbm, zero_hbm, cnt_hbm, out_hbm,
          idxs_v, idxd_v, bufs, cnt_v, accum, gsems, ssems):
        c = lax.axis_index("c")
        s = lax.axis_index("s")
        wid = c * NSUB + s
        pltpu.sync_copy(cnt_hbm, cnt_v)
        pltpu.sync_copy(zero_hbm.at[pl.ds(s * rpt, rpt)],
                        accum.at[pl.ds(s * rpt, rpt)])
        pltpu.sync_copy(src_hbm.at[pl.ds(wid * ncht, ncht)], idxs_v)
        pltpu.sync_copy(dst_hbm.at[pl.ds(wid * ncht, ncht)], idxd_v)
        plsc.subcore_barrier()
        ngrp = cnt_v[...][0] // NBUF
        for b in range(NBUF):
            pltpu.async_copy(tab_hbm.at[idxs_v.at[b]], bufs.at[b], gsems[b])

        def grp(g, carry):
            for b in range(NBUF):
                j = g * NBUF + b
                pltpu.make_async_copy(tab_hbm.at[idxs_v.at[j]], bufs.at[b],
                                      gsems[b]).wait()
                pltpu.async_copy(bufs.at[b], accum.at[idxd_v.at[j]], ssems[b],
                                 add=True)
            for b in range(NBUF):
                jn = (g + 1) * NBUF + b
                pltpu.make_async_copy(bufs.at[b], accum.at[idxd_v.at[jn]],
                                      ssems[b]).wait()
                pltpu.async_copy(tab_hbm.at[idxs_v.at[jn]], bufs.at[b],
                                 gsems[b])
            return carry

        lax.fori_loop(0, ngrp - 1, grp, 0)
        for b in range(NBUF):
            j = (ngrp - 1) * NBUF + b
            pltpu.make_async_copy(tab_hbm.at[idxs_v.at[j]], bufs.at[b],
                                  gsems[b]).wait()
            pltpu.async_copy(bufs.at[b], accum.at[idxd_v.at[j]], ssems[b],
                             add=True)
        for b in range(NBUF):
            pltpu.make_async_copy(bufs.at[b], accum.at[idxd_v.at[0]],
                                  ssems[b]).wait()
        plsc.subcore_barrier()
        pltpu.sync_copy(accum.at[pl.ds(s * rpt, rpt)],
                        out_hbm.at[c, pl.ds(s * rpt, rpt)])

    return k


def _pad_to(a, n, fill):
    return jnp.concatenate(
        [a, jnp.full((n - a.shape[0],), fill, a.dtype)]) if a.shape[0] < n else a


def _tile_layout(a, ncht_eff, ncht_full, fill):
    core = _pad_to(a, NW * ncht_eff * CH, fill).reshape(NW, ncht_eff, CH)
    if ncht_eff == ncht_full:
        return core.reshape(NW * ncht_full, CH)
    pad = jnp.full((NW, ncht_full - ncht_eff, CH), fill, a.dtype)
    return jnp.concatenate([core, pad], axis=1).reshape(NW * ncht_full, CH)


def kernel(x, edge_u_x, edge_u_id, edge_index, train, W_aug, W_c1, b_c1,
           W_c2, b_c2, W_trans, b_trans, w_game, b_game, W_in, b_in,
           W_h1, b_h1, W_h2, b_h2, W_nz, b_nz, W_cl, b_cl):
    nn, d = x.shape
    nu = edge_u_id.shape[0]
    e = edge_index.shape[1]

    npad_u = 4096
    ng = nu * K3
    ngp = -(-ng // (NW * 16)) * (NW * 16)
    ncht_u = -(-(nu * TOPK) // (NW * CH * NBUF)) * NBUF
    ncht_itm = -(-e // (NW * CH * 8)) * 8
    ncht_usr = -(-(e + 2 * ngp) // (NW * CH * 8)) * 8

    zerosD = jnp.zeros((NPAD, D), jnp.float32)
    ones_tab = jnp.concatenate([jnp.ones((nn, D), jnp.float32),
                                jnp.zeros((NT - nn, D), jnp.float32)])
    n_u = jnp.full((16,), ncht_u, jnp.int32)
    n_usr = jnp.full((16,), ncht_usr, jnp.int32)
    n_itm = jnp.full((16,), ncht_itm, jnp.int32)

    agg_k = _make_agg(ncht_usr, NPAD, D)

    xu = x[:nu]
    f_aug = _tc_faug(xu, edge_u_x, W_aug[:d], W_aug[d:])
    idx_u = _tc_topk(edge_u_x, TOPK)
    idx_s = _tc_topk(f_aug, TOPK)

    src_u = _tile_layout(jnp.repeat(jnp.arange(nu, dtype=jnp.int32), TOPK),
                         ncht_u, ncht_usr, ZR)
    dst_u = _tile_layout(idx_u.reshape(-1), ncht_u, ncht_usr, 0)
    dst_s = _tile_layout(idx_s.reshape(-1), ncht_u, ncht_usr, 0)

    cnt_u = agg_k(src_u, dst_u, ones_tab, zerosD, n_u)
    cnt_s = agg_k(src_u, dst_s, ones_tab, zerosD, n_u)
    xs_u, xs_s, rs_u, rs_s = _tc_scaleu(xu, cnt_u, cnt_s, npad_u)
    zpad_u = jnp.zeros((NT - nu, D), jnp.float32)
    p_u = agg_k(src_u, dst_u, jnp.concatenate([xs_u, zpad_u]), zerosD, n_u)
    p_s = agg_k(src_u, dst_s, jnp.concatenate([xs_s, zpad_u]), zerosD, n_u)

    wg8 = jnp.zeros((D, 8), jnp.float32)
    wg8 = wg8.at[:, 0].set(w_game[:d]).at[:, 1].set(w_game[d:])
    x_fuse, ab = _tc_fuse(p_u, p_s, xs_u, xs_s, rs_u, rs_s,
                          W_c1, b_c1, W_c2, b_c2, W_trans, b_trans, wg8)

    idx_g, sA_g, sB_g = _tc_topk_gate(x_fuse, ab[:, 0] + b_game, ab[:, 1],
                                      K3, ZR)
    gs = _pad_to(jnp.repeat(jnp.arange(nu, dtype=jnp.int32), K3), ngp, 0)
    gd = _pad_to(idx_g.reshape(-1), ngp, 0)
    sA = _pad_to(sA_g.reshape(-1), ngp, ZR)
    sB = _pad_to(sB_g.reshape(-1), ngp, ZR)

    src_usr = _tile_layout(jnp.concatenate([edge_index[1], sA, sB]),
                           ncht_usr, ncht_usr, ZR)
    dst_usr = _tile_layout(jnp.concatenate([edge_index[0], gd, gs]),
                           ncht_usr, ncht_usr, 0)
    inA = edge_index[1] < nu + NPAD
    src_iA = _tile_layout(jnp.where(inA, edge_index[0], ZR),
                          ncht_itm, ncht_usr, ZR)
    dst_iA = _tile_layout(jnp.where(inA, edge_index[1] - nu, 0),
                          ncht_itm, ncht_usr, 0)
    src_iB = _tile_layout(jnp.where(inA, ZR, edge_index[0]),
                          ncht_itm, ncht_usr, ZR)
    dst_iB = _tile_layout(jnp.where(inA, 0, edge_index[1] - nu - NPAD),
                          ncht_itm, ncht_usr, 0)

    cnt_usr = agg_k(src_usr, dst_usr, ones_tab, zerosD, n_usr)
    cnt_iA = agg_k(src_iA, dst_iA, ones_tab, zerosD, n_itm)
    cnt_iB = agg_k(src_iB, dst_iB, ones_tab, zerosD, n_itm)
    xs0, rs_b = _tc_prep(x, x_fuse, cnt_usr, cnt_iA, cnt_iB)

    h = xs0
    for w, b in ((W_in, b_in), (W_h1, b_h1), (W_h2, b_h2)):
        pu = agg_k(src_usr, dst_usr, h, zerosD, n_usr)
        pa = agg_k(src_iA, dst_iA, h, zerosD, n_itm)
        pb = agg_k(src_iB, dst_iB, h, zerosD, n_itm)
        h = _tc_layer(pu, pa, pb, h, rs_b, w, b, nu=nu)
    pu = agg_k(src_usr, dst_usr, h, zerosD, n_usr)
    pa = agg_k(src_iA, dst_iA, h, zerosD, n_itm)
    pb = agg_k(src_iB, dst_iB, h, zerosD, n_itm)
    h = _tc_layer(pu, pa, pb, h, rs_b, W_nz, b_nz, nu=nu)
    h = jnp.concatenate([h, jnp.zeros((NT, D - 64), jnp.float32)], axis=1)
    w_cl_pad = jnp.concatenate([W_cl, jnp.zeros((D - 64, W_cl.shape[1]),
                                                jnp.float32)])
    pu = agg_k(src_usr, dst_usr, h, zerosD, n_usr)
    pa = agg_k(src_iA, dst_iA, h, zerosD, n_itm)
    pb = agg_k(src_iB, dst_iB, h, zerosD, n_itm)
    x_out = _tc_layer(pu, pa, pb, h, rs_b, w_cl_pad, b_cl, final=True, nu=nu)

    return (x_out, x_fuse, jnp.zeros(()))
